# baseline (speedup 1.0000x reference)
import functools

import jax
import jax.numpy as jnp
from jax import lax
from jax.experimental import pallas as pl
from jax.experimental.pallas import tpu as pltpu
from jax.experimental.pallas import tpu_sc as plsc

H = 128
EH = 16
N = 10000
E = 160000
NC, NS, LANES = 2, 16, 16
NT = NC * NS
EB = 128
NBLK = 40
EPAD = NT * NBLK * EB
NACC = 10112
ZR = NACC // NS
RB = 1000
GRID = N // RB
SQ = 1.0 / float(H) ** 0.5


@functools.lru_cache(maxsize=None)
def _sc_kernels():
    mesh = plsc.VectorSubcoreMesh(core_axis_name="c", subcore_axis_name="s",
                                  num_cores=NC, num_subcores=NS)

    @functools.partial(
        pl.kernel,
        out_type=jax.ShapeDtypeStruct((NC, NACC, H), jnp.float32),
        mesh=mesh,
        scratch_types=[
            pltpu.VMEM((NBLK, EB), jnp.int32),
            pltpu.VMEM((NBLK, EB), jnp.int32),
            pltpu.VMEM((2, EB, H), jnp.float32),
            pltpu.VMEM_SHARED((NACC, H), jnp.float32),
            pltpu.SemaphoreType.DMA,
            pltpu.SemaphoreType.DMA,
            pltpu.SemaphoreType.DMA,
            pltpu.SemaphoreType.DMA,
        ],
    )
    def apply_A(x_hbm, src_hbm, dst_hbm, zeros_hbm, out_hbm,
                src_v, dst_v, rows_v, acc, g0, g1, s0, s1):
        gsem = (g0, g1)
        ssem = (s0, s1)
        cid = lax.axis_index("c")
        sid = lax.axis_index("s")
        gid = cid * NS + sid
        pltpu.sync_copy(zeros_hbm.at[pl.ds(sid * ZR, ZR)],
                        acc.at[pl.ds(sid * ZR, ZR)])
        pltpu.sync_copy(src_hbm.at[gid], src_v)
        pltpu.sync_copy(dst_hbm.at[gid], dst_v)
        plsc.subcore_barrier()

        def gwait(j):
            pltpu.make_async_copy(x_hbm.at[src_v.at[0]], rows_v.at[j],
                                  gsem[j]).wait()

        def swait(j):
            pltpu.make_async_copy(rows_v.at[j], acc.at[dst_v.at[0]],
                                  ssem[j]).wait()

        for j in range(2):
            pltpu.async_copy(x_hbm.at[src_v.at[j]], rows_v.at[j], gsem[j])

        def rnd(r, carry):
            b = 2 * r
            for j in range(2):
                gwait(j)
                pltpu.async_copy(rows_v.at[j], acc.at[dst_v.at[b + j]],
                                 ssem[j], add=True)
            for j in range(2):
                swait(j)
                pltpu.async_copy(x_hbm.at[src_v.at[b + 2 + j]], rows_v.at[j],
                                 gsem[j])
            return carry

        lax.fori_loop(0, NBLK // 2 - 1, rnd, 0)
        for j in range(2):
            gwait(j)
            pltpu.async_copy(rows_v.at[j], acc.at[dst_v.at[NBLK - 2 + j]],
                             ssem[j], add=True)
        for j in range(2):
            swait(j)
        plsc.subcore_barrier()
        pltpu.sync_copy(acc.at[pl.ds(sid * ZR, ZR)],
                        out_hbm.at[cid, pl.ds(sid * ZR, ZR)])

    @functools.partial(
        pl.kernel,
        out_type=jax.ShapeDtypeStruct((NC, NACC, EH), jnp.float32),
        mesh=mesh,
        compiler_params=pltpu.CompilerParams(use_tc_tiling_on_sc=False),
        scratch_types=[
            pltpu.VMEM((NBLK, EB), jnp.int32),
            pltpu.VMEM((NBLK * EB, EH), jnp.float32),
            pltpu.VMEM_SHARED((NACC, EH), jnp.float32),
        ],
    )
    def edge_agg(ea_hbm, dst_hbm, zeros_hbm, out_hbm, dst_v, ea_v, acc):
        cid = lax.axis_index("c")
        sid = lax.axis_index("s")
        gid = cid * NS + sid
        pltpu.sync_copy(zeros_hbm.at[pl.ds(sid * ZR, ZR)],
                        acc.at[pl.ds(sid * ZR, ZR)])
        pltpu.sync_copy(dst_hbm.at[gid], dst_v)
        pltpu.sync_copy(ea_hbm.at[gid], ea_v)
        plsc.subcore_barrier()

        def step(b, carry):
            pltpu.sync_copy(ea_v.at[pl.ds(b * EB, EB)], acc.at[dst_v.at[b]],
                            add=True)
            return carry

        lax.fori_loop(0, NBLK, step, 0)
        plsc.subcore_barrier()
        pltpu.sync_copy(acc.at[pl.ds(sid * ZR, ZR)],
                        out_hbm.at[cid, pl.ds(sid * ZR, ZR)])

    return apply_A, edge_agg


def _dot(a, b):
    return jnp.dot(a, b, preferred_element_type=jnp.float32)


def _dmpnn_body(ax_ref, e_ref, x_ref, wma, wmb, wih, whh, bih, bhh, out_ref):
    m = _dot(ax_ref[0] + ax_ref[1], wma[...]) + _dot(e_ref[0] + e_ref[1], wmb[...])
    gi = _dot(m, wih[...]) + bih[...]
    gh = _dot(x_ref[...], whh[...]) + bhh[...]
    r = jax.nn.sigmoid(gi[:, :H] + gh[:, :H])
    z = jax.nn.sigmoid(gi[:, H:2 * H] + gh[:, H:2 * H])
    n = jnp.tanh(gi[:, 2 * H:] + r * gh[:, 2 * H:])
    out_ref[...] = (1.0 - z) * n + z * x_ref[...]


def _dmpnn(ax, eagg, x, p):
    wma, wmb = p["W_msg"][:H], p["W_msg"][H:]
    return pl.pallas_call(
        _dmpnn_body,
        grid=(GRID,),
        in_specs=[
            pl.BlockSpec((NC, RB, H), lambda i: (0, i, 0)),
            pl.BlockSpec((NC, RB, EH), lambda i: (0, i, 0)),
            pl.BlockSpec((RB, H), lambda i: (i, 0)),
            pl.BlockSpec((H, H), lambda i: (0, 0)),
            pl.BlockSpec((EH, H), lambda i: (0, 0)),
            pl.BlockSpec((H, 3 * H), lambda i: (0, 0)),
            pl.BlockSpec((H, 3 * H), lambda i: (0, 0)),
            pl.BlockSpec((1, 3 * H), lambda i: (0, 0)),
            pl.BlockSpec((1, 3 * H), lambda i: (0, 0)),
        ],
        out_specs=pl.BlockSpec((RB, H), lambda i: (i, 0)),
        out_shape=jax.ShapeDtypeStruct((N, H), jnp.float32),
    )(ax, eagg, x, wma, wmb, p["Wih"], p["Whh"],
      p["bih"].reshape(1, -1), p["bhh"].reshape(1, -1))


def _mlp(base, w1, b1, w2, b2):
    return _dot(jax.nn.relu(_dot(base, w1[...]) + b1[...]), w2[...]) + b2[...]


def _gin3_body(h_ref, u_ref, w11, b11, w21, b21, w12, b12, w22, b22,
               w13, b13, w23, b23, t1_ref, s2_ref, s3_ref):
    base = h_ref[...] + u_ref[0] + u_ref[1]
    t1_ref[...] = _mlp(base, w11, b11, w21, b21)
    s2_ref[...] = _mlp(base, w12, b12, w22, b22)
    s3_ref[...] = _mlp(base, w13, b13, w23, b23)


def _gin3(h2, u, convs):
    wspec = pl.BlockSpec((H, H), lambda i: (0, 0))
    bspec = pl.BlockSpec((1, H), lambda i: (0, 0))
    args = []
    for c in convs:
        args += [c["W1"], c["b1"].reshape(1, -1), c["W2"], c["b2"].reshape(1, -1)]
    return pl.pallas_call(
        _gin3_body,
        grid=(GRID,),
        in_specs=[
            pl.BlockSpec((RB, H), lambda i: (i, 0)),
            pl.BlockSpec((NC, RB, H), lambda i: (0, i, 0)),
        ] + [wspec, bspec, wspec, bspec] * 3,
        out_specs=[pl.BlockSpec((RB, H), lambda i: (i, 0))] * 3,
        out_shape=[jax.ShapeDtypeStruct((N, H), jnp.float32)] * 3,
    )(h2, u, *args)


def _gin2_body(s2_ref, a2_ref, s3_ref, a3_ref, w12, b12, w22, b22,
               w13, b13, w23, b23, t2_ref, s3p_ref):
    t2_ref[...] = _mlp(s2_ref[...] + a2_ref[0] + a2_ref[1], w12, b12, w22, b22)
    s3p_ref[...] = _mlp(s3_ref[...] + a3_ref[0] + a3_ref[1], w13, b13, w23, b23)


def _gin2(s2, a2, s3, a3, c2, c3):
    wspec = pl.BlockSpec((H, H), lambda i: (0, 0))
    bspec = pl.BlockSpec((1, H), lambda i: (0, 0))
    return pl.pallas_call(
        _gin2_body,
        grid=(GRID,),
        in_specs=[
            pl.BlockSpec((RB, H), lambda i: (i, 0)),
            pl.BlockSpec((NC, RB, H), lambda i: (0, i, 0)),
            pl.BlockSpec((RB, H), lambda i: (i, 0)),
            pl.BlockSpec((NC, RB, H), lambda i: (0, i, 0)),
            wspec, bspec, wspec, bspec, wspec, bspec, wspec, bspec,
        ],
        out_specs=[pl.BlockSpec((RB, H), lambda i: (i, 0))] * 2,
        out_shape=[jax.ShapeDtypeStruct((N, H), jnp.float32)] * 2,
    )(s2, a2, s3, a3, c2["W1"], c2["b1"].reshape(1, -1), c2["W2"],
      c2["b2"].reshape(1, -1), c3["W1"], c3["b1"].reshape(1, -1), c3["W2"],
      c3["b2"].reshape(1, -1))


def _mix_body(t1_ref, t2_ref, s3p_ref, a3_ref, w13, b13, w23, b23,
              wout, bout, g1, gb1, g2r, gb2, mask_ref,
              hmix_ref, s_ref):
    i = pl.program_id(0)
    t3 = _mlp(s3p_ref[...] + a3_ref[0] + a3_ref[1], w13, b13, w23, b23)
    cat = jnp.concatenate([t1_ref[...], t2_ref[...], t3], axis=1)
    hm = _dot(cat, wout[...]) + bout[...]
    hmix_ref[...] = hm
    a1 = jax.nn.relu(_dot(hm, g1[...]) + gb1[...])
    logit = jnp.sum(a1 * g2r[...], axis=1, keepdims=True) + gb2[...]
    w = jax.nn.sigmoid(logit) * mask_ref[...]
    part = jnp.sum(w * hm, axis=0, keepdims=True)

    @pl.when(i == 0)
    def _():
        s_ref[...] = part

    @pl.when(i > 0)
    def _():
        s_ref[...] = s_ref[...] + part


def _mix(t1, t2, s3p, a3, c3, pmix, psk, mask_f):
    wspec = pl.BlockSpec((H, H), lambda i: (0, 0))
    bspec = pl.BlockSpec((1, H), lambda i: (0, 0))
    return pl.pallas_call(
        _mix_body,
        grid=(GRID,),
        in_specs=[
            pl.BlockSpec((RB, H), lambda i: (i, 0)),
            pl.BlockSpec((RB, H), lambda i: (i, 0)),
            pl.BlockSpec((RB, H), lambda i: (i, 0)),
            pl.BlockSpec((NC, RB, H), lambda i: (0, i, 0)),
            wspec, bspec, wspec, bspec,
            pl.BlockSpec((3 * H, H), lambda i: (0, 0)), bspec,
            pl.BlockSpec((H, H // 2), lambda i: (0, 0)),
            pl.BlockSpec((1, H // 2), lambda i: (0, 0)),
            pl.BlockSpec((1, H // 2), lambda i: (0, 0)),
            pl.BlockSpec((1, 1), lambda i: (0, 0)),
            pl.BlockSpec((RB, 1), lambda i: (i, 0)),
        ],
        out_specs=[pl.BlockSpec((RB, H), lambda i: (i, 0)),
                   pl.BlockSpec((1, H), lambda i: (0, 0))],
        out_shape=[jax.ShapeDtypeStruct((N, H), jnp.float32),
                   jax.ShapeDtypeStruct((1, H), jnp.float32)],
    )(t1, t2, s3p, a3, c3["W1"], c3["b1"].reshape(1, -1), c3["W2"],
      c3["b2"].reshape(1, -1), pmix["Wout"], pmix["bout"].reshape(1, -1),
      psk["g1"], psk["gb1"].reshape(1, -1), psk["g2"].reshape(1, -1),
      psk["gb2"].reshape(1, 1), mask_f)


def _att_body(hmix_ref, s_ref, rc_ref, si_ref, skw, skwih, skwhh, skbih, skbhh,
              wq, bq, wk, bk, wv, bv, wo, bo, out_ref,
              hs_s, kq_s, acc_s, smem):
    i = pl.program_id(0)

    @pl.when(i == 0)
    def _():
        m_total = _dot(s_ref[...] + rc_ref[...], skw[...])
        gi = _dot(m_total, skwih[...]) + skbih[...]
        gh = _dot(si_ref[...], skwhh[...]) + skbhh[...]
        r = jax.nn.sigmoid(gi[:, :H] + gh[:, :H])
        z = jax.nn.sigmoid(gi[:, H:2 * H] + gh[:, H:2 * H])
        n = jnp.tanh(gi[:, 2 * H:] + r * gh[:, 2 * H:])
        hs = (1.0 - z) * n + z * si_ref[...]
        hs_s[...] = hs
        q = _dot(hs, wq[...]) + bq[...]
        kq = lax.dot_general(q, wk[...], (((1,), (1,)), ((), ())),
                             preferred_element_type=jnp.float32)
        kq_s[...] = kq
        smem[2] = jnp.sum(bk[...] * q)
        k_rc = _dot(rc_ref[...], wk[...]) + bk[...]
        smem[3] = jnp.sum(k_rc * q) * SQ
        k_s = _dot(hs, wk[...]) + bk[...]
        smem[4] = jnp.sum(k_s * q) * SQ
        smem[0] = -1e30
        smem[1] = 0.0
        acc_s[...] = jnp.zeros_like(acc_s)

    hm = hmix_ref[...]
    sb = (jnp.sum(hm * kq_s[...], axis=1, keepdims=True) + smem[2]) * SQ
    m_old = smem[0]
    m_new = jnp.maximum(m_old, jnp.max(sb))
    scale = jnp.exp(m_old - m_new)
    p = jnp.exp(sb - m_new)
    smem[0] = m_new
    smem[1] = smem[1] * scale + jnp.sum(p)
    acc_s[...] = acc_s[...] * scale + jnp.sum(p * hm, axis=0, keepdims=True)

    @pl.when(i == GRID - 1)
    def _():
        m_old2 = smem[0]
        m_fin = jnp.maximum(jnp.maximum(m_old2, smem[3]), smem[4])
        sc = jnp.exp(m_old2 - m_fin)
        p_rc = jnp.exp(smem[3] - m_fin)
        p_s = jnp.exp(smem[4] - m_fin)
        denom = smem[1] * sc + p_rc + p_s
        ctx = (acc_s[...] * sc + p_rc * rc_ref[...] + p_s * hs_s[...]) / denom
        att = _dot(_dot(ctx, wv[...]) + bv[...], wo[...]) + bo[...] + hs_s[...]
        out_ref[...] = jnp.maximum(att, 0.0)


def _att(hmix, s_sum, psk, patt, rc_init, s_init):
    wspec = pl.BlockSpec((H, H), lambda i: (0, 0))
    bspec = pl.BlockSpec((1, H), lambda i: (0, 0))
    return pl.pallas_call(
        _att_body,
        grid=(GRID,),
        in_specs=[
            pl.BlockSpec((RB, H), lambda i: (i, 0)),
            bspec, bspec, bspec,
            wspec,
            pl.BlockSpec((H, 3 * H), lambda i: (0, 0)),
            pl.BlockSpec((H, 3 * H), lambda i: (0, 0)),
            pl.BlockSpec((1, 3 * H), lambda i: (0, 0)),
            pl.BlockSpec((1, 3 * H), lambda i: (0, 0)),
            wspec, bspec, wspec, bspec, wspec, bspec, wspec, bspec,
        ],
        out_specs=pl.BlockSpec((1, H), lambda i: (0, 0)),
        out_shape=jax.ShapeDtypeStruct((1, H), jnp.float32),
        scratch_shapes=[
            pltpu.VMEM((1, H), jnp.float32),
            pltpu.VMEM((1, H), jnp.float32),
            pltpu.VMEM((1, H), jnp.float32),
            pltpu.SMEM((8,), jnp.float32),
        ],
    )(hmix, s_sum, rc_init, s_init, psk["W"], psk["Wih"], psk["Whh"],
      psk["bih"].reshape(1, -1), psk["bhh"].reshape(1, -1),
      patt["Wq"], patt["bq"].reshape(1, -1), patt["Wk"],
      patt["bk"].reshape(1, -1), patt["Wv"], patt["bv"].reshape(1, -1),
      patt["Wo"], patt["bo"].reshape(1, -1))


def kernel(x, edge_index, edge_attr, rc_mask, params):
    src = edge_index[0]
    dst = edge_index[1]
    pad = EPAD - E
    src3 = jnp.concatenate([src, jnp.zeros((pad,), jnp.int32)]).reshape(NT, NBLK, EB)
    dst3 = jnp.concatenate([dst, jnp.full((pad,), N, jnp.int32)]).reshape(NT, NBLK, EB)
    ea3 = jnp.concatenate([edge_attr, jnp.zeros((pad, EH), jnp.float32)]
                          ).reshape(NT, NBLK * EB, EH)
    zeros_h = jnp.zeros((NACC, H), jnp.float32)
    zeros_e = jnp.zeros((NACC, EH), jnp.float32)
    mask_f = (~rc_mask).astype(jnp.float32).reshape(N, 1)

    _apply_A, _edge_agg = _sc_kernels()
    eagg = _edge_agg(ea3, dst3, zeros_e)
    ax = _apply_A(x, src3, dst3, zeros_h)
    h1 = _dmpnn(ax, eagg, x, params["gnn1"])
    ah1 = _apply_A(h1, src3, dst3, zeros_h)
    h2 = _dmpnn(ah1, eagg, h1, params["gnn2"])
    ah2 = _apply_A(h2, src3, dst3, zeros_h)
    convs = params["mix"]["convs"]
    t1, s2, s3 = _gin3(h2, ah2, convs)
    as2 = _apply_A(s2, src3, dst3, zeros_h)
    as3 = _apply_A(s3, src3, dst3, zeros_h)
    t2, s3p = _gin2(s2, as2, s3, as3, convs[1], convs[2])
    as3p = _apply_A(s3p, src3, dst3, zeros_h)
    hmix, s_sum = _mix(t1, t2, s3p, as3p, convs[2], params["mix"],
                       params["skip"], mask_f)
    out = _att(hmix, s_sum, params["skip"], params["att"],
               params["rc_init"], params["s_init"])
    return out.reshape(H)


# speedup vs baseline: 1.0076x; 1.0076x over previous
import functools

import jax
import jax.numpy as jnp
from jax import lax
from jax.experimental import pallas as pl
from jax.experimental.pallas import tpu as pltpu
from jax.experimental.pallas import tpu_sc as plsc

H = 128
EH = 16
N = 10000
E = 160000
NC, NS, LANES = 2, 16, 16
NT = NC * NS
EB = 128
NBLK = 40
EPAD = NT * NBLK * EB
NACC = 10112
ZR = NACC // NS
RB = 1000
GRID = N // RB
SQ = 1.0 / float(H) ** 0.5


@functools.lru_cache(maxsize=None)
def _sc_kernels():
    mesh = plsc.VectorSubcoreMesh(core_axis_name="c", subcore_axis_name="s",
                                  num_cores=NC, num_subcores=NS)

    @functools.partial(
        pl.kernel,
        out_type=jax.ShapeDtypeStruct((NC, NACC, H), jnp.float32),
        mesh=mesh,
        scratch_types=[
            pltpu.VMEM((NBLK, EB), jnp.int32),
            pltpu.VMEM((NBLK, EB), jnp.int32),
            pltpu.VMEM((2, EB, H), jnp.float32),
            pltpu.VMEM_SHARED((NACC, H), jnp.float32),
            pltpu.SemaphoreType.DMA,
            pltpu.SemaphoreType.DMA,
            pltpu.SemaphoreType.DMA,
            pltpu.SemaphoreType.DMA,
            pltpu.SemaphoreType.DMA,
        ],
    )
    def apply_A(x_hbm, src_hbm, dst_hbm, zeros_hbm, out_hbm,
                src_v, dst_v, rows_v, acc, g0, g1, s0, s1, zsem):
        gsem = (g0, g1)
        ssem = (s0, s1)
        cid = lax.axis_index("c")
        sid = lax.axis_index("s")
        gid = cid * NS + sid
        pltpu.async_copy(zeros_hbm.at[pl.ds(sid * ZR, ZR)],
                         acc.at[pl.ds(sid * ZR, ZR)], zsem)
        pltpu.sync_copy(src_hbm.at[gid], src_v)
        pltpu.sync_copy(dst_hbm.at[gid], dst_v)
        def gwait(j):
            pltpu.make_async_copy(x_hbm.at[src_v.at[0]], rows_v.at[j],
                                  gsem[j]).wait()

        def swait(j):
            pltpu.make_async_copy(rows_v.at[j], acc.at[dst_v.at[0]],
                                  ssem[j]).wait()

        for j in range(2):
            pltpu.async_copy(x_hbm.at[src_v.at[j]], rows_v.at[j], gsem[j])
        pltpu.make_async_copy(zeros_hbm.at[pl.ds(sid * ZR, ZR)],
                              acc.at[pl.ds(sid * ZR, ZR)], zsem).wait()
        plsc.subcore_barrier()

        def rnd(r, carry):
            b = 2 * r
            for j in range(2):
                gwait(j)
                pltpu.async_copy(rows_v.at[j], acc.at[dst_v.at[b + j]],
                                 ssem[j], add=True)
            for j in range(2):
                swait(j)
                pltpu.async_copy(x_hbm.at[src_v.at[b + 2 + j]], rows_v.at[j],
                                 gsem[j])
            return carry

        lax.fori_loop(0, NBLK // 2 - 1, rnd, 0)
        for j in range(2):
            gwait(j)
            pltpu.async_copy(rows_v.at[j], acc.at[dst_v.at[NBLK - 2 + j]],
                             ssem[j], add=True)
        for j in range(2):
            swait(j)
        plsc.subcore_barrier()
        pltpu.sync_copy(acc.at[pl.ds(sid * ZR, ZR)],
                        out_hbm.at[cid, pl.ds(sid * ZR, ZR)])

    @functools.partial(
        pl.kernel,
        out_type=jax.ShapeDtypeStruct((NC, NACC, EH), jnp.float32),
        mesh=mesh,
        compiler_params=pltpu.CompilerParams(use_tc_tiling_on_sc=False),
        scratch_types=[
            pltpu.VMEM((NBLK, EB), jnp.int32),
            pltpu.VMEM((NBLK * EB, EH), jnp.float32),
            pltpu.VMEM_SHARED((NACC, EH), jnp.float32),
            pltpu.SemaphoreType.DMA,
            pltpu.SemaphoreType.DMA,
            pltpu.SemaphoreType.DMA,
        ],
    )
    def edge_agg(ea_hbm, dst_hbm, zeros_hbm, out_hbm, dst_v, ea_v, acc,
                 s0, s1, zsem):
        ssem = (s0, s1)
        cid = lax.axis_index("c")
        sid = lax.axis_index("s")
        gid = cid * NS + sid
        pltpu.async_copy(zeros_hbm.at[pl.ds(sid * ZR, ZR)],
                         acc.at[pl.ds(sid * ZR, ZR)], zsem)
        pltpu.sync_copy(dst_hbm.at[gid], dst_v)
        pltpu.sync_copy(ea_hbm.at[gid], ea_v)
        pltpu.make_async_copy(zeros_hbm.at[pl.ds(sid * ZR, ZR)],
                              acc.at[pl.ds(sid * ZR, ZR)], zsem).wait()
        plsc.subcore_barrier()

        def swait(j):
            pltpu.make_async_copy(ea_v.at[pl.ds(0, EB)], acc.at[dst_v.at[0]],
                                  ssem[j]).wait()

        for j in range(2):
            pltpu.async_copy(ea_v.at[pl.ds(j * EB, EB)], acc.at[dst_v.at[j]],
                             ssem[j], add=True)

        def step(r, carry):
            b = 2 * r
            for j in range(2):
                swait(j)
                pltpu.async_copy(ea_v.at[pl.ds((b + 2 + j) * EB, EB)],
                                 acc.at[dst_v.at[b + 2 + j]], ssem[j],
                                 add=True)
            return carry

        lax.fori_loop(0, NBLK // 2 - 1, step, 0)
        for j in range(2):
            swait(j)
        plsc.subcore_barrier()
        pltpu.sync_copy(acc.at[pl.ds(sid * ZR, ZR)],
                        out_hbm.at[cid, pl.ds(sid * ZR, ZR)])

    return apply_A, edge_agg


def _dot(a, b):
    return jnp.dot(a, b, preferred_element_type=jnp.float32)


def _dmpnn_body(ax_ref, e_ref, x_ref, wma, wmb, wih, whh, bih, bhh, out_ref):
    m = _dot(ax_ref[0] + ax_ref[1], wma[...]) + _dot(e_ref[0] + e_ref[1], wmb[...])
    gi = _dot(m, wih[...]) + bih[...]
    gh = _dot(x_ref[...], whh[...]) + bhh[...]
    r = jax.nn.sigmoid(gi[:, :H] + gh[:, :H])
    z = jax.nn.sigmoid(gi[:, H:2 * H] + gh[:, H:2 * H])
    n = jnp.tanh(gi[:, 2 * H:] + r * gh[:, 2 * H:])
    out_ref[...] = (1.0 - z) * n + z * x_ref[...]


def _dmpnn(ax, eagg, x, p):
    wma, wmb = p["W_msg"][:H], p["W_msg"][H:]
    return pl.pallas_call(
        _dmpnn_body,
        grid=(GRID,),
        in_specs=[
            pl.BlockSpec((NC, RB, H), lambda i: (0, i, 0)),
            pl.BlockSpec((NC, RB, EH), lambda i: (0, i, 0)),
            pl.BlockSpec((RB, H), lambda i: (i, 0)),
            pl.BlockSpec((H, H), lambda i: (0, 0)),
            pl.BlockSpec((EH, H), lambda i: (0, 0)),
            pl.BlockSpec((H, 3 * H), lambda i: (0, 0)),
            pl.BlockSpec((H, 3 * H), lambda i: (0, 0)),
            pl.BlockSpec((1, 3 * H), lambda i: (0, 0)),
            pl.BlockSpec((1, 3 * H), lambda i: (0, 0)),
        ],
        out_specs=pl.BlockSpec((RB, H), lambda i: (i, 0)),
        out_shape=jax.ShapeDtypeStruct((N, H), jnp.float32),
    )(ax, eagg, x, wma, wmb, p["Wih"], p["Whh"],
      p["bih"].reshape(1, -1), p["bhh"].reshape(1, -1))


def _mlp(base, w1, b1, w2, b2):
    return _dot(jax.nn.relu(_dot(base, w1[...]) + b1[...]), w2[...]) + b2[...]


def _gin3_body(h_ref, u_ref, w11, b11, w21, b21, w12, b12, w22, b22,
               w13, b13, w23, b23, t1_ref, s2_ref, s3_ref):
    base = h_ref[...] + u_ref[0] + u_ref[1]
    t1_ref[...] = _mlp(base, w11, b11, w21, b21)
    s2_ref[...] = _mlp(base, w12, b12, w22, b22)
    s3_ref[...] = _mlp(base, w13, b13, w23, b23)


def _gin3(h2, u, convs):
    wspec = pl.BlockSpec((H, H), lambda i: (0, 0))
    bspec = pl.BlockSpec((1, H), lambda i: (0, 0))
    args = []
    for c in convs:
        args += [c["W1"], c["b1"].reshape(1, -1), c["W2"], c["b2"].reshape(1, -1)]
    return pl.pallas_call(
        _gin3_body,
        grid=(GRID,),
        in_specs=[
            pl.BlockSpec((RB, H), lambda i: (i, 0)),
            pl.BlockSpec((NC, RB, H), lambda i: (0, i, 0)),
        ] + [wspec, bspec, wspec, bspec] * 3,
        out_specs=[pl.BlockSpec((RB, H), lambda i: (i, 0))] * 3,
        out_shape=[jax.ShapeDtypeStruct((N, H), jnp.float32)] * 3,
    )(h2, u, *args)


def _gin2_body(s2_ref, a2_ref, s3_ref, a3_ref, w12, b12, w22, b22,
               w13, b13, w23, b23, t2_ref, s3p_ref):
    t2_ref[...] = _mlp(s2_ref[...] + a2_ref[0] + a2_ref[1], w12, b12, w22, b22)
    s3p_ref[...] = _mlp(s3_ref[...] + a3_ref[0] + a3_ref[1], w13, b13, w23, b23)


def _gin2(s2, a2, s3, a3, c2, c3):
    wspec = pl.BlockSpec((H, H), lambda i: (0, 0))
    bspec = pl.BlockSpec((1, H), lambda i: (0, 0))
    return pl.pallas_call(
        _gin2_body,
        grid=(GRID,),
        in_specs=[
            pl.BlockSpec((RB, H), lambda i: (i, 0)),
            pl.BlockSpec((NC, RB, H), lambda i: (0, i, 0)),
            pl.BlockSpec((RB, H), lambda i: (i, 0)),
            pl.BlockSpec((NC, RB, H), lambda i: (0, i, 0)),
            wspec, bspec, wspec, bspec, wspec, bspec, wspec, bspec,
        ],
        out_specs=[pl.BlockSpec((RB, H), lambda i: (i, 0))] * 2,
        out_shape=[jax.ShapeDtypeStruct((N, H), jnp.float32)] * 2,
    )(s2, a2, s3, a3, c2["W1"], c2["b1"].reshape(1, -1), c2["W2"],
      c2["b2"].reshape(1, -1), c3["W1"], c3["b1"].reshape(1, -1), c3["W2"],
      c3["b2"].reshape(1, -1))


def _mix_body(t1_ref, t2_ref, s3p_ref, a3_ref, w13, b13, w23, b23,
              wout, bout, g1, gb1, g2r, gb2, mask_ref,
              hmix_ref, s_ref):
    i = pl.program_id(0)
    t3 = _mlp(s3p_ref[...] + a3_ref[0] + a3_ref[1], w13, b13, w23, b23)
    cat = jnp.concatenate([t1_ref[...], t2_ref[...], t3], axis=1)
    hm = _dot(cat, wout[...]) + bout[...]
    hmix_ref[...] = hm
    a1 = jax.nn.relu(_dot(hm, g1[...]) + gb1[...])
    logit = jnp.sum(a1 * g2r[...], axis=1, keepdims=True) + gb2[...]
    w = jax.nn.sigmoid(logit) * mask_ref[...]
    part = jnp.sum(w * hm, axis=0, keepdims=True)

    @pl.when(i == 0)
    def _():
        s_ref[...] = part

    @pl.when(i > 0)
    def _():
        s_ref[...] = s_ref[...] + part


def _mix(t1, t2, s3p, a3, c3, pmix, psk, mask_f):
    wspec = pl.BlockSpec((H, H), lambda i: (0, 0))
    bspec = pl.BlockSpec((1, H), lambda i: (0, 0))
    return pl.pallas_call(
        _mix_body,
        grid=(GRID,),
        in_specs=[
            pl.BlockSpec((RB, H), lambda i: (i, 0)),
            pl.BlockSpec((RB, H), lambda i: (i, 0)),
            pl.BlockSpec((RB, H), lambda i: (i, 0)),
            pl.BlockSpec((NC, RB, H), lambda i: (0, i, 0)),
            wspec, bspec, wspec, bspec,
            pl.BlockSpec((3 * H, H), lambda i: (0, 0)), bspec,
            pl.BlockSpec((H, H // 2), lambda i: (0, 0)),
            pl.BlockSpec((1, H // 2), lambda i: (0, 0)),
            pl.BlockSpec((1, H // 2), lambda i: (0, 0)),
            pl.BlockSpec((1, 1), lambda i: (0, 0)),
            pl.BlockSpec((RB, 1), lambda i: (i, 0)),
        ],
        out_specs=[pl.BlockSpec((RB, H), lambda i: (i, 0)),
                   pl.BlockSpec((1, H), lambda i: (0, 0))],
        out_shape=[jax.ShapeDtypeStruct((N, H), jnp.float32),
                   jax.ShapeDtypeStruct((1, H), jnp.float32)],
    )(t1, t2, s3p, a3, c3["W1"], c3["b1"].reshape(1, -1), c3["W2"],
      c3["b2"].reshape(1, -1), pmix["Wout"], pmix["bout"].reshape(1, -1),
      psk["g1"], psk["gb1"].reshape(1, -1), psk["g2"].reshape(1, -1),
      psk["gb2"].reshape(1, 1), mask_f)


def _att_body(hmix_ref, s_ref, rc_ref, si_ref, skw, skwih, skwhh, skbih, skbhh,
              wq, bq, wk, bk, wv, bv, wo, bo, out_ref,
              hs_s, kq_s, acc_s, smem):
    i = pl.program_id(0)

    @pl.when(i == 0)
    def _():
        m_total = _dot(s_ref[...] + rc_ref[...], skw[...])
        gi = _dot(m_total, skwih[...]) + skbih[...]
        gh = _dot(si_ref[...], skwhh[...]) + skbhh[...]
        r = jax.nn.sigmoid(gi[:, :H] + gh[:, :H])
        z = jax.nn.sigmoid(gi[:, H:2 * H] + gh[:, H:2 * H])
        n = jnp.tanh(gi[:, 2 * H:] + r * gh[:, 2 * H:])
        hs = (1.0 - z) * n + z * si_ref[...]
        hs_s[...] = hs
        q = _dot(hs, wq[...]) + bq[...]
        kq = lax.dot_general(q, wk[...], (((1,), (1,)), ((), ())),
                             preferred_element_type=jnp.float32)
        kq_s[...] = kq
        smem[2] = jnp.sum(bk[...] * q)
        k_rc = _dot(rc_ref[...], wk[...]) + bk[...]
        smem[3] = jnp.sum(k_rc * q) * SQ
        k_s = _dot(hs, wk[...]) + bk[...]
        smem[4] = jnp.sum(k_s * q) * SQ
        smem[0] = -1e30
        smem[1] = 0.0
        acc_s[...] = jnp.zeros_like(acc_s)

    hm = hmix_ref[...]
    sb = (jnp.sum(hm * kq_s[...], axis=1, keepdims=True) + smem[2]) * SQ
    m_old = smem[0]
    m_new = jnp.maximum(m_old, jnp.max(sb))
    scale = jnp.exp(m_old - m_new)
    p = jnp.exp(sb - m_new)
    smem[0] = m_new
    smem[1] = smem[1] * scale + jnp.sum(p)
    acc_s[...] = acc_s[...] * scale + jnp.sum(p * hm, axis=0, keepdims=True)

    @pl.when(i == GRID - 1)
    def _():
        m_old2 = smem[0]
        m_fin = jnp.maximum(jnp.maximum(m_old2, smem[3]), smem[4])
        sc = jnp.exp(m_old2 - m_fin)
        p_rc = jnp.exp(smem[3] - m_fin)
        p_s = jnp.exp(smem[4] - m_fin)
        denom = smem[1] * sc + p_rc + p_s
        ctx = (acc_s[...] * sc + p_rc * rc_ref[...] + p_s * hs_s[...]) / denom
        att = _dot(_dot(ctx, wv[...]) + bv[...], wo[...]) + bo[...] + hs_s[...]
        out_ref[...] = jnp.maximum(att, 0.0)


def _att(hmix, s_sum, psk, patt, rc_init, s_init):
    wspec = pl.BlockSpec((H, H), lambda i: (0, 0))
    bspec = pl.BlockSpec((1, H), lambda i: (0, 0))
    return pl.pallas_call(
        _att_body,
        grid=(GRID,),
        in_specs=[
            pl.BlockSpec((RB, H), lambda i: (i, 0)),
            bspec, bspec, bspec,
            wspec,
            pl.BlockSpec((H, 3 * H), lambda i: (0, 0)),
            pl.BlockSpec((H, 3 * H), lambda i: (0, 0)),
            pl.BlockSpec((1, 3 * H), lambda i: (0, 0)),
            pl.BlockSpec((1, 3 * H), lambda i: (0, 0)),
            wspec, bspec, wspec, bspec, wspec, bspec, wspec, bspec,
        ],
        out_specs=pl.BlockSpec((1, H), lambda i: (0, 0)),
        out_shape=jax.ShapeDtypeStruct((1, H), jnp.float32),
        scratch_shapes=[
            pltpu.VMEM((1, H), jnp.float32),
            pltpu.VMEM((1, H), jnp.float32),
            pltpu.VMEM((1, H), jnp.float32),
            pltpu.SMEM((8,), jnp.float32),
        ],
    )(hmix, s_sum, rc_init, s_init, psk["W"], psk["Wih"], psk["Whh"],
      psk["bih"].reshape(1, -1), psk["bhh"].reshape(1, -1),
      patt["Wq"], patt["bq"].reshape(1, -1), patt["Wk"],
      patt["bk"].reshape(1, -1), patt["Wv"], patt["bv"].reshape(1, -1),
      patt["Wo"], patt["bo"].reshape(1, -1))


def kernel(x, edge_index, edge_attr, rc_mask, params):
    src = edge_index[0]
    dst = edge_index[1]
    pad = EPAD - E
    src3 = jnp.concatenate([src, jnp.zeros((pad,), jnp.int32)]).reshape(NT, NBLK, EB)
    dst3 = jnp.concatenate([dst, jnp.full((pad,), N, jnp.int32)]).reshape(NT, NBLK, EB)
    ea3 = jnp.concatenate([edge_attr, jnp.zeros((pad, EH), jnp.float32)]
                          ).reshape(NT, NBLK * EB, EH)
    zeros_h = jnp.zeros((NACC, H), jnp.float32)
    zeros_e = jnp.zeros((NACC, EH), jnp.float32)
    mask_f = (~rc_mask).astype(jnp.float32).reshape(N, 1)

    _apply_A, _edge_agg = _sc_kernels()
    eagg = _edge_agg(ea3, dst3, zeros_e)
    ax = _apply_A(x, src3, dst3, zeros_h)
    h1 = _dmpnn(ax, eagg, x, params["gnn1"])
    ah1 = _apply_A(h1, src3, dst3, zeros_h)
    h2 = _dmpnn(ah1, eagg, h1, params["gnn2"])
    ah2 = _apply_A(h2, src3, dst3, zeros_h)
    convs = params["mix"]["convs"]
    t1, s2, s3 = _gin3(h2, ah2, convs)
    as2 = _apply_A(s2, src3, dst3, zeros_h)
    as3 = _apply_A(s3, src3, dst3, zeros_h)
    t2, s3p = _gin2(s2, as2, s3, as3, convs[1], convs[2])
    as3p = _apply_A(s3p, src3, dst3, zeros_h)
    hmix, s_sum = _mix(t1, t2, s3p, as3p, convs[2], params["mix"],
                       params["skip"], mask_f)
    out = _att(hmix, s_sum, params["skip"], params["att"],
               params["rc_init"], params["s_init"])
    return out.reshape(H)


# speedup vs baseline: 1.0336x; 1.0258x over previous
import functools

import jax
import jax.numpy as jnp
from jax import lax
from jax.experimental import pallas as pl
from jax.experimental.pallas import tpu as pltpu
from jax.experimental.pallas import tpu_sc as plsc

H = 128
EH = 16
N = 10000
E = 160000
NC, NS, LANES = 2, 16, 16
NT = NC * NS
EB = 128
NBLK = 40
EPAD = NT * NBLK * EB
NACC = 10112
ZR = NACC // NS
RB = 1000
GRID = N // RB
SQ = 1.0 / float(H) ** 0.5


@functools.lru_cache(maxsize=None)
def _sc_kernels():
    mesh = plsc.VectorSubcoreMesh(core_axis_name="c", subcore_axis_name="s",
                                  num_cores=NC, num_subcores=NS)

    @functools.partial(
        pl.kernel,
        out_type=jax.ShapeDtypeStruct((NC, NACC, H), jnp.float32),
        mesh=mesh,
        scratch_types=[
            pltpu.VMEM((NBLK, EB), jnp.int32),
            pltpu.VMEM((NBLK, EB), jnp.int32),
            pltpu.VMEM((2, EB, H), jnp.float32),
            pltpu.VMEM_SHARED((NACC, H), jnp.float32),
            pltpu.SemaphoreType.DMA,
            pltpu.SemaphoreType.DMA,
            pltpu.SemaphoreType.DMA,
            pltpu.SemaphoreType.DMA,
            pltpu.SemaphoreType.DMA,
        ],
    )
    def apply_A(x_hbm, src_hbm, dst_hbm, zeros_hbm, out_hbm,
                src_v, dst_v, rows_v, acc, g0, g1, s0, s1, zsem):
        gsem = (g0, g1)
        ssem = (s0, s1)
        cid = lax.axis_index("c")
        sid = lax.axis_index("s")
        gid = cid * NS + sid
        pltpu.async_copy(zeros_hbm.at[pl.ds(sid * ZR, ZR)],
                         acc.at[pl.ds(sid * ZR, ZR)], zsem)
        pltpu.sync_copy(src_hbm.at[gid], src_v)
        pltpu.sync_copy(dst_hbm.at[gid], dst_v)
        def gwait(j):
            pltpu.make_async_copy(x_hbm.at[src_v.at[0]], rows_v.at[j],
                                  gsem[j]).wait()

        for j in range(2):
            pltpu.async_copy(x_hbm.at[src_v.at[j]], rows_v.at[j], gsem[j])
        pltpu.make_async_copy(zeros_hbm.at[pl.ds(sid * ZR, ZR)],
                              acc.at[pl.ds(sid * ZR, ZR)], zsem).wait()
        plsc.subcore_barrier()

        def rnd(r, carry):
            b = 2 * r
            for j in range(2):
                gwait(j)
                pltpu.sync_copy(rows_v.at[j], acc.at[dst_v.at[b + j]],
                                add=True)
                pltpu.async_copy(x_hbm.at[src_v.at[b + 2 + j]], rows_v.at[j],
                                 gsem[j])
            return carry

        lax.fori_loop(0, NBLK // 2 - 1, rnd, 0)
        for j in range(2):
            gwait(j)
            pltpu.sync_copy(rows_v.at[j], acc.at[dst_v.at[NBLK - 2 + j]],
                            add=True)
        plsc.subcore_barrier()
        pltpu.sync_copy(acc.at[pl.ds(sid * ZR, ZR)],
                        out_hbm.at[cid, pl.ds(sid * ZR, ZR)])

    @functools.partial(
        pl.kernel,
        out_type=jax.ShapeDtypeStruct((NC, NACC, EH), jnp.float32),
        mesh=mesh,
        compiler_params=pltpu.CompilerParams(use_tc_tiling_on_sc=False),
        scratch_types=[
            pltpu.VMEM((NBLK, EB), jnp.int32),
            pltpu.VMEM((NBLK * EB, EH), jnp.float32),
            pltpu.VMEM_SHARED((NACC, EH), jnp.float32),
            pltpu.SemaphoreType.DMA,
            pltpu.SemaphoreType.DMA,
            pltpu.SemaphoreType.DMA,
        ],
    )
    def edge_agg(ea_hbm, dst_hbm, zeros_hbm, out_hbm, dst_v, ea_v, acc,
                 s0, s1, zsem):
        ssem = (s0, s1)
        cid = lax.axis_index("c")
        sid = lax.axis_index("s")
        gid = cid * NS + sid
        pltpu.async_copy(zeros_hbm.at[pl.ds(sid * ZR, ZR)],
                         acc.at[pl.ds(sid * ZR, ZR)], zsem)
        pltpu.sync_copy(dst_hbm.at[gid], dst_v)
        pltpu.sync_copy(ea_hbm.at[gid], ea_v)
        pltpu.make_async_copy(zeros_hbm.at[pl.ds(sid * ZR, ZR)],
                              acc.at[pl.ds(sid * ZR, ZR)], zsem).wait()
        plsc.subcore_barrier()

        def swait(j):
            pltpu.make_async_copy(ea_v.at[pl.ds(0, EB)], acc.at[dst_v.at[0]],
                                  ssem[j]).wait()

        for j in range(2):
            pltpu.async_copy(ea_v.at[pl.ds(j * EB, EB)], acc.at[dst_v.at[j]],
                             ssem[j], add=True)

        def step(r, carry):
            b = 2 * r
            for j in range(2):
                swait(j)
                pltpu.async_copy(ea_v.at[pl.ds((b + 2 + j) * EB, EB)],
                                 acc.at[dst_v.at[b + 2 + j]], ssem[j],
                                 add=True)
            return carry

        lax.fori_loop(0, NBLK // 2 - 1, step, 0)
        for j in range(2):
            swait(j)
        plsc.subcore_barrier()
        pltpu.sync_copy(acc.at[pl.ds(sid * ZR, ZR)],
                        out_hbm.at[cid, pl.ds(sid * ZR, ZR)])

    return apply_A, edge_agg


def _dot(a, b):
    return jnp.dot(a, b, preferred_element_type=jnp.float32)


def _dmpnn_body(ax_ref, e_ref, x_ref, wma, wmb, wih, whh, bih, bhh, out_ref):
    m = _dot(ax_ref[0] + ax_ref[1], wma[...]) + _dot(e_ref[0] + e_ref[1], wmb[...])
    gi = _dot(m, wih[...]) + bih[...]
    gh = _dot(x_ref[...], whh[...]) + bhh[...]
    r = jax.nn.sigmoid(gi[:, :H] + gh[:, :H])
    z = jax.nn.sigmoid(gi[:, H:2 * H] + gh[:, H:2 * H])
    n = jnp.tanh(gi[:, 2 * H:] + r * gh[:, 2 * H:])
    out_ref[...] = (1.0 - z) * n + z * x_ref[...]


def _dmpnn(ax, eagg, x, p):
    wma, wmb = p["W_msg"][:H], p["W_msg"][H:]
    return pl.pallas_call(
        _dmpnn_body,
        grid=(GRID,),
        in_specs=[
            pl.BlockSpec((NC, RB, H), lambda i: (0, i, 0)),
            pl.BlockSpec((NC, RB, EH), lambda i: (0, i, 0)),
            pl.BlockSpec((RB, H), lambda i: (i, 0)),
            pl.BlockSpec((H, H), lambda i: (0, 0)),
            pl.BlockSpec((EH, H), lambda i: (0, 0)),
            pl.BlockSpec((H, 3 * H), lambda i: (0, 0)),
            pl.BlockSpec((H, 3 * H), lambda i: (0, 0)),
            pl.BlockSpec((1, 3 * H), lambda i: (0, 0)),
            pl.BlockSpec((1, 3 * H), lambda i: (0, 0)),
        ],
        out_specs=pl.BlockSpec((RB, H), lambda i: (i, 0)),
        out_shape=jax.ShapeDtypeStruct((N, H), jnp.float32),
    )(ax, eagg, x, wma, wmb, p["Wih"], p["Whh"],
      p["bih"].reshape(1, -1), p["bhh"].reshape(1, -1))


def _mlp(base, w1, b1, w2, b2):
    return _dot(jax.nn.relu(_dot(base, w1[...]) + b1[...]), w2[...]) + b2[...]


def _gin3_body(h_ref, u_ref, w11, b11, w21, b21, w12, b12, w22, b22,
               w13, b13, w23, b23, t1_ref, s2_ref, s3_ref):
    base = h_ref[...] + u_ref[0] + u_ref[1]
    t1_ref[...] = _mlp(base, w11, b11, w21, b21)
    s2_ref[...] = _mlp(base, w12, b12, w22, b22)
    s3_ref[...] = _mlp(base, w13, b13, w23, b23)


def _gin3(h2, u, convs):
    wspec = pl.BlockSpec((H, H), lambda i: (0, 0))
    bspec = pl.BlockSpec((1, H), lambda i: (0, 0))
    args = []
    for c in convs:
        args += [c["W1"], c["b1"].reshape(1, -1), c["W2"], c["b2"].reshape(1, -1)]
    return pl.pallas_call(
        _gin3_body,
        grid=(GRID,),
        in_specs=[
            pl.BlockSpec((RB, H), lambda i: (i, 0)),
            pl.BlockSpec((NC, RB, H), lambda i: (0, i, 0)),
        ] + [wspec, bspec, wspec, bspec] * 3,
        out_specs=[pl.BlockSpec((RB, H), lambda i: (i, 0))] * 3,
        out_shape=[jax.ShapeDtypeStruct((N, H), jnp.float32)] * 3,
    )(h2, u, *args)


def _gin2_body(s2_ref, a2_ref, s3_ref, a3_ref, w12, b12, w22, b22,
               w13, b13, w23, b23, t2_ref, s3p_ref):
    t2_ref[...] = _mlp(s2_ref[...] + a2_ref[0] + a2_ref[1], w12, b12, w22, b22)
    s3p_ref[...] = _mlp(s3_ref[...] + a3_ref[0] + a3_ref[1], w13, b13, w23, b23)


def _gin2(s2, a2, s3, a3, c2, c3):
    wspec = pl.BlockSpec((H, H), lambda i: (0, 0))
    bspec = pl.BlockSpec((1, H), lambda i: (0, 0))
    return pl.pallas_call(
        _gin2_body,
        grid=(GRID,),
        in_specs=[
            pl.BlockSpec((RB, H), lambda i: (i, 0)),
            pl.BlockSpec((NC, RB, H), lambda i: (0, i, 0)),
            pl.BlockSpec((RB, H), lambda i: (i, 0)),
            pl.BlockSpec((NC, RB, H), lambda i: (0, i, 0)),
            wspec, bspec, wspec, bspec, wspec, bspec, wspec, bspec,
        ],
        out_specs=[pl.BlockSpec((RB, H), lambda i: (i, 0))] * 2,
        out_shape=[jax.ShapeDtypeStruct((N, H), jnp.float32)] * 2,
    )(s2, a2, s3, a3, c2["W1"], c2["b1"].reshape(1, -1), c2["W2"],
      c2["b2"].reshape(1, -1), c3["W1"], c3["b1"].reshape(1, -1), c3["W2"],
      c3["b2"].reshape(1, -1))


def _mix_body(t1_ref, t2_ref, s3p_ref, a3_ref, w13, b13, w23, b23,
              wout, bout, g1, gb1, g2r, gb2, mask_ref,
              hmix_ref, s_ref):
    i = pl.program_id(0)
    t3 = _mlp(s3p_ref[...] + a3_ref[0] + a3_ref[1], w13, b13, w23, b23)
    cat = jnp.concatenate([t1_ref[...], t2_ref[...], t3], axis=1)
    hm = _dot(cat, wout[...]) + bout[...]
    hmix_ref[...] = hm
    a1 = jax.nn.relu(_dot(hm, g1[...]) + gb1[...])
    logit = jnp.sum(a1 * g2r[...], axis=1, keepdims=True) + gb2[...]
    w = jax.nn.sigmoid(logit) * mask_ref[...]
    part = jnp.sum(w * hm, axis=0, keepdims=True)

    @pl.when(i == 0)
    def _():
        s_ref[...] = part

    @pl.when(i > 0)
    def _():
        s_ref[...] = s_ref[...] + part


def _mix(t1, t2, s3p, a3, c3, pmix, psk, mask_f):
    wspec = pl.BlockSpec((H, H), lambda i: (0, 0))
    bspec = pl.BlockSpec((1, H), lambda i: (0, 0))
    return pl.pallas_call(
        _mix_body,
        grid=(GRID,),
        in_specs=[
            pl.BlockSpec((RB, H), lambda i: (i, 0)),
            pl.BlockSpec((RB, H), lambda i: (i, 0)),
            pl.BlockSpec((RB, H), lambda i: (i, 0)),
            pl.BlockSpec((NC, RB, H), lambda i: (0, i, 0)),
            wspec, bspec, wspec, bspec,
            pl.BlockSpec((3 * H, H), lambda i: (0, 0)), bspec,
            pl.BlockSpec((H, H // 2), lambda i: (0, 0)),
            pl.BlockSpec((1, H // 2), lambda i: (0, 0)),
            pl.BlockSpec((1, H // 2), lambda i: (0, 0)),
            pl.BlockSpec((1, 1), lambda i: (0, 0)),
            pl.BlockSpec((RB, 1), lambda i: (i, 0)),
        ],
        out_specs=[pl.BlockSpec((RB, H), lambda i: (i, 0)),
                   pl.BlockSpec((1, H), lambda i: (0, 0))],
        out_shape=[jax.ShapeDtypeStruct((N, H), jnp.float32),
                   jax.ShapeDtypeStruct((1, H), jnp.float32)],
    )(t1, t2, s3p, a3, c3["W1"], c3["b1"].reshape(1, -1), c3["W2"],
      c3["b2"].reshape(1, -1), pmix["Wout"], pmix["bout"].reshape(1, -1),
      psk["g1"], psk["gb1"].reshape(1, -1), psk["g2"].reshape(1, -1),
      psk["gb2"].reshape(1, 1), mask_f)


def _att_body(hmix_ref, s_ref, rc_ref, si_ref, skw, skwih, skwhh, skbih, skbhh,
              wq, bq, wk, bk, wv, bv, wo, bo, out_ref,
              hs_s, kq_s, acc_s, smem):
    i = pl.program_id(0)

    @pl.when(i == 0)
    def _():
        m_total = _dot(s_ref[...] + rc_ref[...], skw[...])
        gi = _dot(m_total, skwih[...]) + skbih[...]
        gh = _dot(si_ref[...], skwhh[...]) + skbhh[...]
        r = jax.nn.sigmoid(gi[:, :H] + gh[:, :H])
        z = jax.nn.sigmoid(gi[:, H:2 * H] + gh[:, H:2 * H])
        n = jnp.tanh(gi[:, 2 * H:] + r * gh[:, 2 * H:])
        hs = (1.0 - z) * n + z * si_ref[...]
        hs_s[...] = hs
        q = _dot(hs, wq[...]) + bq[...]
        kq = lax.dot_general(q, wk[...], (((1,), (1,)), ((), ())),
                             preferred_element_type=jnp.float32)
        kq_s[...] = kq
        smem[2] = jnp.sum(bk[...] * q)
        k_rc = _dot(rc_ref[...], wk[...]) + bk[...]
        smem[3] = jnp.sum(k_rc * q) * SQ
        k_s = _dot(hs, wk[...]) + bk[...]
        smem[4] = jnp.sum(k_s * q) * SQ
        smem[0] = -1e30
        smem[1] = 0.0
        acc_s[...] = jnp.zeros_like(acc_s)

    hm = hmix_ref[...]
    sb = (jnp.sum(hm * kq_s[...], axis=1, keepdims=True) + smem[2]) * SQ
    m_old = smem[0]
    m_new = jnp.maximum(m_old, jnp.max(sb))
    scale = jnp.exp(m_old - m_new)
    p = jnp.exp(sb - m_new)
    smem[0] = m_new
    smem[1] = smem[1] * scale + jnp.sum(p)
    acc_s[...] = acc_s[...] * scale + jnp.sum(p * hm, axis=0, keepdims=True)

    @pl.when(i == GRID - 1)
    def _():
        m_old2 = smem[0]
        m_fin = jnp.maximum(jnp.maximum(m_old2, smem[3]), smem[4])
        sc = jnp.exp(m_old2 - m_fin)
        p_rc = jnp.exp(smem[3] - m_fin)
        p_s = jnp.exp(smem[4] - m_fin)
        denom = smem[1] * sc + p_rc + p_s
        ctx = (acc_s[...] * sc + p_rc * rc_ref[...] + p_s * hs_s[...]) / denom
        att = _dot(_dot(ctx, wv[...]) + bv[...], wo[...]) + bo[...] + hs_s[...]
        out_ref[...] = jnp.maximum(att, 0.0)


def _att(hmix, s_sum, psk, patt, rc_init, s_init):
    wspec = pl.BlockSpec((H, H), lambda i: (0, 0))
    bspec = pl.BlockSpec((1, H), lambda i: (0, 0))
    return pl.pallas_call(
        _att_body,
        grid=(GRID,),
        in_specs=[
            pl.BlockSpec((RB, H), lambda i: (i, 0)),
            bspec, bspec, bspec,
            wspec,
            pl.BlockSpec((H, 3 * H), lambda i: (0, 0)),
            pl.BlockSpec((H, 3 * H), lambda i: (0, 0)),
            pl.BlockSpec((1, 3 * H), lambda i: (0, 0)),
            pl.BlockSpec((1, 3 * H), lambda i: (0, 0)),
            wspec, bspec, wspec, bspec, wspec, bspec, wspec, bspec,
        ],
        out_specs=pl.BlockSpec((1, H), lambda i: (0, 0)),
        out_shape=jax.ShapeDtypeStruct((1, H), jnp.float32),
        scratch_shapes=[
            pltpu.VMEM((1, H), jnp.float32),
            pltpu.VMEM((1, H), jnp.float32),
            pltpu.VMEM((1, H), jnp.float32),
            pltpu.SMEM((8,), jnp.float32),
        ],
    )(hmix, s_sum, rc_init, s_init, psk["W"], psk["Wih"], psk["Whh"],
      psk["bih"].reshape(1, -1), psk["bhh"].reshape(1, -1),
      patt["Wq"], patt["bq"].reshape(1, -1), patt["Wk"],
      patt["bk"].reshape(1, -1), patt["Wv"], patt["bv"].reshape(1, -1),
      patt["Wo"], patt["bo"].reshape(1, -1))


def kernel(x, edge_index, edge_attr, rc_mask, params):
    src = edge_index[0]
    dst = edge_index[1]
    pad = EPAD - E
    src3 = jnp.concatenate([src, jnp.zeros((pad,), jnp.int32)]).reshape(NT, NBLK, EB)
    dst3 = jnp.concatenate([dst, jnp.full((pad,), N, jnp.int32)]).reshape(NT, NBLK, EB)
    ea3 = jnp.concatenate([edge_attr, jnp.zeros((pad, EH), jnp.float32)]
                          ).reshape(NT, NBLK * EB, EH)
    zeros_h = jnp.zeros((NACC, H), jnp.float32)
    zeros_e = jnp.zeros((NACC, EH), jnp.float32)
    mask_f = (~rc_mask).astype(jnp.float32).reshape(N, 1)

    _apply_A, _edge_agg = _sc_kernels()
    eagg = _edge_agg(ea3, dst3, zeros_e)
    ax = _apply_A(x, src3, dst3, zeros_h)
    h1 = _dmpnn(ax, eagg, x, params["gnn1"])
    ah1 = _apply_A(h1, src3, dst3, zeros_h)
    h2 = _dmpnn(ah1, eagg, h1, params["gnn2"])
    ah2 = _apply_A(h2, src3, dst3, zeros_h)
    convs = params["mix"]["convs"]
    t1, s2, s3 = _gin3(h2, ah2, convs)
    as2 = _apply_A(s2, src3, dst3, zeros_h)
    as3 = _apply_A(s3, src3, dst3, zeros_h)
    t2, s3p = _gin2(s2, as2, s3, as3, convs[1], convs[2])
    as3p = _apply_A(s3p, src3, dst3, zeros_h)
    hmix, s_sum = _mix(t1, t2, s3p, as3p, convs[2], params["mix"],
                       params["skip"], mask_f)
    out = _att(hmix, s_sum, params["skip"], params["att"],
               params["rc_init"], params["s_init"])
    return out.reshape(H)


# speedup vs baseline: 1.0350x; 1.0014x over previous
import functools

import jax
import jax.numpy as jnp
from jax import lax
from jax.experimental import pallas as pl
from jax.experimental.pallas import tpu as pltpu
from jax.experimental.pallas import tpu_sc as plsc

H = 128
EH = 16
N = 10000
E = 160000
NC, NS, LANES = 2, 16, 16
NT = NC * NS
EB = 128
NBLK = 40
EPAD = NT * NBLK * EB
NACC = 10112
ZR = NACC // NS
RB = 1000
GRID = N // RB
SQ = 1.0 / float(H) ** 0.5


@functools.lru_cache(maxsize=None)
def _sc_kernels():
    mesh = plsc.VectorSubcoreMesh(core_axis_name="c", subcore_axis_name="s",
                                  num_cores=NC, num_subcores=NS)

    @functools.partial(
        pl.kernel,
        out_type=jax.ShapeDtypeStruct((NC, NACC, H), jnp.float32),
        mesh=mesh,
        scratch_types=[
            pltpu.VMEM((NBLK, EB), jnp.int32),
            pltpu.VMEM((NBLK, EB), jnp.int32),
            pltpu.VMEM((2, EB, H), jnp.float32),
            pltpu.VMEM_SHARED((NACC, H), jnp.float32),
            pltpu.SemaphoreType.DMA,
            pltpu.SemaphoreType.DMA,
            pltpu.SemaphoreType.DMA,
            pltpu.SemaphoreType.DMA,
            pltpu.SemaphoreType.DMA,
        ],
    )
    def apply_A(x_hbm, src_hbm, dst_hbm, zeros_hbm, out_hbm,
                src_v, dst_v, rows_v, acc, g0, g1, s0, s1, zsem):
        gsem = (g0, g1)
        ssem = (s0, s1)
        cid = lax.axis_index("c")
        sid = lax.axis_index("s")
        gid = cid * NS + sid
        pltpu.async_copy(zeros_hbm.at[pl.ds(sid * ZR, ZR)],
                         acc.at[pl.ds(sid * ZR, ZR)], zsem)
        pltpu.sync_copy(src_hbm.at[gid], src_v)
        pltpu.sync_copy(dst_hbm.at[gid], dst_v)
        def gwait(j):
            pltpu.make_async_copy(x_hbm.at[src_v.at[0]], rows_v.at[j],
                                  gsem[j]).wait()

        for j in range(2):
            pltpu.async_copy(x_hbm.at[src_v.at[j]], rows_v.at[j], gsem[j])
        pltpu.make_async_copy(zeros_hbm.at[pl.ds(sid * ZR, ZR)],
                              acc.at[pl.ds(sid * ZR, ZR)], zsem).wait()
        plsc.subcore_barrier()

        def rnd(r, carry):
            b = 2 * r
            for j in range(2):
                gwait(j)
                pltpu.sync_copy(rows_v.at[j], acc.at[dst_v.at[b + j]],
                                add=True)
                pltpu.async_copy(x_hbm.at[src_v.at[b + 2 + j]], rows_v.at[j],
                                 gsem[j])
            return carry

        lax.fori_loop(0, NBLK // 2 - 1, rnd, 0)
        for j in range(2):
            gwait(j)
            pltpu.sync_copy(rows_v.at[j], acc.at[dst_v.at[NBLK - 2 + j]],
                            add=True)
        plsc.subcore_barrier()
        pltpu.sync_copy(acc.at[pl.ds(sid * ZR, ZR)],
                        out_hbm.at[cid, pl.ds(sid * ZR, ZR)])

    @functools.partial(
        pl.kernel,
        out_type=jax.ShapeDtypeStruct((NC, NACC, EH), jnp.float32),
        mesh=mesh,
        compiler_params=pltpu.CompilerParams(use_tc_tiling_on_sc=False),
        scratch_types=[
            pltpu.VMEM((NBLK, EB), jnp.int32),
            pltpu.VMEM((NBLK * EB, EH), jnp.float32),
            pltpu.VMEM_SHARED((NACC, EH), jnp.float32),
            pltpu.SemaphoreType.DMA,
            pltpu.SemaphoreType.DMA,
            pltpu.SemaphoreType.DMA,
        ],
    )
    def edge_agg(ea_hbm, dst_hbm, zeros_hbm, out_hbm, dst_v, ea_v, acc,
                 s0, s1, zsem):
        ssem = (s0, s1)
        cid = lax.axis_index("c")
        sid = lax.axis_index("s")
        gid = cid * NS + sid
        pltpu.async_copy(zeros_hbm.at[pl.ds(sid * ZR, ZR)],
                         acc.at[pl.ds(sid * ZR, ZR)], zsem)
        pltpu.sync_copy(dst_hbm.at[gid], dst_v)
        pltpu.sync_copy(ea_hbm.at[gid], ea_v)
        pltpu.make_async_copy(zeros_hbm.at[pl.ds(sid * ZR, ZR)],
                              acc.at[pl.ds(sid * ZR, ZR)], zsem).wait()
        plsc.subcore_barrier()

        def swait(j):
            pltpu.make_async_copy(ea_v.at[pl.ds(0, EB)], acc.at[dst_v.at[0]],
                                  ssem[j]).wait()

        for j in range(2):
            pltpu.async_copy(ea_v.at[pl.ds(j * EB, EB)], acc.at[dst_v.at[j]],
                             ssem[j], add=True)

        def step(r, carry):
            b = 2 * r
            for j in range(2):
                swait(j)
                pltpu.async_copy(ea_v.at[pl.ds((b + 2 + j) * EB, EB)],
                                 acc.at[dst_v.at[b + 2 + j]], ssem[j],
                                 add=True)
            return carry

        lax.fori_loop(0, NBLK // 2 - 1, step, 0)
        for j in range(2):
            swait(j)
        plsc.subcore_barrier()
        pltpu.sync_copy(acc.at[pl.ds(sid * ZR, ZR)],
                        out_hbm.at[cid, pl.ds(sid * ZR, ZR)])

    return apply_A, edge_agg


def _dot(a, b):
    return jnp.dot(a, b, preferred_element_type=jnp.float32)


def _dmpnn_body(ax_ref, e_ref, x_ref, wma, wmb, wih, whh, bih, bhh, out_ref):
    m = _dot(ax_ref[0] + ax_ref[1], wma[...]) + _dot(e_ref[0] + e_ref[1], wmb[...])
    gi = _dot(m, wih[...]) + bih[...]
    gh = _dot(x_ref[...], whh[...]) + bhh[...]
    r = jax.nn.sigmoid(gi[:, :H] + gh[:, :H])
    z = jax.nn.sigmoid(gi[:, H:2 * H] + gh[:, H:2 * H])
    n = jnp.tanh(gi[:, 2 * H:] + r * gh[:, 2 * H:])
    out_ref[...] = (1.0 - z) * n + z * x_ref[...]


def _dmpnn(ax, eagg, x, p):
    wma, wmb = p["W_msg"][:H], p["W_msg"][H:]
    return pl.pallas_call(
        _dmpnn_body,
        grid=(GRID,),
        in_specs=[
            pl.BlockSpec((NC, RB, H), lambda i: (0, i, 0)),
            pl.BlockSpec((NC, RB, EH), lambda i: (0, i, 0)),
            pl.BlockSpec((RB, H), lambda i: (i, 0)),
            pl.BlockSpec((H, H), lambda i: (0, 0)),
            pl.BlockSpec((EH, H), lambda i: (0, 0)),
            pl.BlockSpec((H, 3 * H), lambda i: (0, 0)),
            pl.BlockSpec((H, 3 * H), lambda i: (0, 0)),
            pl.BlockSpec((1, 3 * H), lambda i: (0, 0)),
            pl.BlockSpec((1, 3 * H), lambda i: (0, 0)),
        ],
        out_specs=pl.BlockSpec((RB, H), lambda i: (i, 0)),
        out_shape=jax.ShapeDtypeStruct((N, H), jnp.float32),
    )(ax, eagg, x, wma, wmb, p["Wih"], p["Whh"],
      p["bih"].reshape(1, -1), p["bhh"].reshape(1, -1))


def _mlp(base, w1, b1, w2, b2):
    return _dot(jax.nn.relu(_dot(base, w1[...]) + b1[...]), w2[...]) + b2[...]


def _gin3_body(h_ref, u_ref, w11, b11, w21, b21, w12, b12, w22, b22,
               w13, b13, w23, b23, t1_ref, s2_ref, s3_ref):
    base = h_ref[...] + u_ref[0] + u_ref[1]
    t1_ref[...] = _mlp(base, w11, b11, w21, b21)
    s2_ref[...] = _mlp(base, w12, b12, w22, b22)
    s3_ref[...] = _mlp(base, w13, b13, w23, b23)


def _gin3(h2, u, convs):
    wspec = pl.BlockSpec((H, H), lambda i: (0, 0))
    bspec = pl.BlockSpec((1, H), lambda i: (0, 0))
    args = []
    for c in convs:
        args += [c["W1"], c["b1"].reshape(1, -1), c["W2"], c["b2"].reshape(1, -1)]
    return pl.pallas_call(
        _gin3_body,
        grid=(GRID,),
        in_specs=[
            pl.BlockSpec((RB, H), lambda i: (i, 0)),
            pl.BlockSpec((NC, RB, H), lambda i: (0, i, 0)),
        ] + [wspec, bspec, wspec, bspec] * 3,
        out_specs=[pl.BlockSpec((RB, H), lambda i: (i, 0))] * 3,
        out_shape=[jax.ShapeDtypeStruct((N, H), jnp.float32)] * 3,
    )(h2, u, *args)


def _gin2_body(s2_ref, a2_ref, s3_ref, a3_ref, w12, b12, w22, b22,
               w13, b13, w23, b23, t2_ref, s3p_ref):
    t2_ref[...] = _mlp(s2_ref[...] + a2_ref[0] + a2_ref[1], w12, b12, w22, b22)
    s3p_ref[...] = _mlp(s3_ref[...] + a3_ref[0] + a3_ref[1], w13, b13, w23, b23)


def _gin2(s2, a2, s3, a3, c2, c3):
    wspec = pl.BlockSpec((H, H), lambda i: (0, 0))
    bspec = pl.BlockSpec((1, H), lambda i: (0, 0))
    return pl.pallas_call(
        _gin2_body,
        grid=(GRID,),
        in_specs=[
            pl.BlockSpec((RB, H), lambda i: (i, 0)),
            pl.BlockSpec((NC, RB, H), lambda i: (0, i, 0)),
            pl.BlockSpec((RB, H), lambda i: (i, 0)),
            pl.BlockSpec((NC, RB, H), lambda i: (0, i, 0)),
            wspec, bspec, wspec, bspec, wspec, bspec, wspec, bspec,
        ],
        out_specs=[pl.BlockSpec((RB, H), lambda i: (i, 0))] * 2,
        out_shape=[jax.ShapeDtypeStruct((N, H), jnp.float32)] * 2,
    )(s2, a2, s3, a3, c2["W1"], c2["b1"].reshape(1, -1), c2["W2"],
      c2["b2"].reshape(1, -1), c3["W1"], c3["b1"].reshape(1, -1), c3["W2"],
      c3["b2"].reshape(1, -1))


def _mix_body(t1_ref, t2_ref, s3p_ref, a3_ref, w13, b13, w23, b23,
              wout, bout, g1, gb1, g2r, gb2, mask_ref,
              hmix_ref, s_ref):
    i = pl.program_id(0)
    t3 = _mlp(s3p_ref[...] + a3_ref[0] + a3_ref[1], w13, b13, w23, b23)
    cat = jnp.concatenate([t1_ref[...], t2_ref[...], t3], axis=1)
    hm = _dot(cat, wout[...]) + bout[...]
    hmix_ref[...] = hm
    a1 = jax.nn.relu(_dot(hm, g1[...]) + gb1[...])
    logit = jnp.sum(a1 * g2r[...], axis=1, keepdims=True) + gb2[...]
    w = jax.nn.sigmoid(logit) * mask_ref[...]
    part = jnp.sum(w * hm, axis=0, keepdims=True)

    @pl.when(i == 0)
    def _():
        s_ref[...] = part

    @pl.when(i > 0)
    def _():
        s_ref[...] = s_ref[...] + part


def _mix(t1, t2, s3p, a3, c3, pmix, psk, mask_f):
    wspec = pl.BlockSpec((H, H), lambda i: (0, 0))
    bspec = pl.BlockSpec((1, H), lambda i: (0, 0))
    return pl.pallas_call(
        _mix_body,
        grid=(GRID,),
        in_specs=[
            pl.BlockSpec((RB, H), lambda i: (i, 0)),
            pl.BlockSpec((RB, H), lambda i: (i, 0)),
            pl.BlockSpec((RB, H), lambda i: (i, 0)),
            pl.BlockSpec((NC, RB, H), lambda i: (0, i, 0)),
            wspec, bspec, wspec, bspec,
            pl.BlockSpec((3 * H, H), lambda i: (0, 0)), bspec,
            pl.BlockSpec((H, H // 2), lambda i: (0, 0)),
            pl.BlockSpec((1, H // 2), lambda i: (0, 0)),
            pl.BlockSpec((1, H // 2), lambda i: (0, 0)),
            pl.BlockSpec((1, 1), lambda i: (0, 0)),
            pl.BlockSpec((RB, 1), lambda i: (i, 0)),
        ],
        out_specs=[pl.BlockSpec((RB, H), lambda i: (i, 0)),
                   pl.BlockSpec((1, H), lambda i: (0, 0))],
        out_shape=[jax.ShapeDtypeStruct((N, H), jnp.float32),
                   jax.ShapeDtypeStruct((1, H), jnp.float32)],
    )(t1, t2, s3p, a3, c3["W1"], c3["b1"].reshape(1, -1), c3["W2"],
      c3["b2"].reshape(1, -1), pmix["Wout"], pmix["bout"].reshape(1, -1),
      psk["g1"], psk["gb1"].reshape(1, -1), psk["g2"].reshape(1, -1),
      psk["gb2"].reshape(1, 1), mask_f)


def _att_body(hmix_ref, s_ref, rc_ref, si_ref, skw, skwih, skwhh, skbih, skbhh,
              wq, bq, wk, bk, wv, bv, wo, bo, out_ref,
              hs_s, kq_s, acc_s, smem):
    i = pl.program_id(0)

    @pl.when(i == 0)
    def _():
        m_total = _dot(s_ref[...] + rc_ref[...], skw[...])
        gi = _dot(m_total, skwih[...]) + skbih[...]
        gh = _dot(si_ref[...], skwhh[...]) + skbhh[...]
        r = jax.nn.sigmoid(gi[:, :H] + gh[:, :H])
        z = jax.nn.sigmoid(gi[:, H:2 * H] + gh[:, H:2 * H])
        n = jnp.tanh(gi[:, 2 * H:] + r * gh[:, 2 * H:])
        hs = (1.0 - z) * n + z * si_ref[...]
        hs_s[...] = hs
        q = _dot(hs, wq[...]) + bq[...]
        kq = lax.dot_general(q, wk[...], (((1,), (1,)), ((), ())),
                             preferred_element_type=jnp.float32)
        kq_s[...] = kq
        smem[2] = jnp.sum(bk[...] * q)
        k_rc = _dot(rc_ref[...], wk[...]) + bk[...]
        smem[3] = jnp.sum(k_rc * q) * SQ
        k_s = _dot(hs, wk[...]) + bk[...]
        smem[4] = jnp.sum(k_s * q) * SQ
        smem[0] = -1e30
        smem[1] = 0.0
        acc_s[...] = jnp.zeros_like(acc_s)

    hm = hmix_ref[...]
    sb = (jnp.sum(hm * kq_s[...], axis=1, keepdims=True) + smem[2]) * SQ
    m_old = smem[0]
    m_new = jnp.maximum(m_old, jnp.max(sb))
    scale = jnp.exp(m_old - m_new)
    p = jnp.exp(sb - m_new)
    smem[0] = m_new
    smem[1] = smem[1] * scale + jnp.sum(p)
    acc_s[...] = acc_s[...] * scale + jnp.sum(p * hm, axis=0, keepdims=True)

    @pl.when(i == GRID - 1)
    def _():
        m_old2 = smem[0]
        m_fin = jnp.maximum(jnp.maximum(m_old2, smem[3]), smem[4])
        sc = jnp.exp(m_old2 - m_fin)
        p_rc = jnp.exp(smem[3] - m_fin)
        p_s = jnp.exp(smem[4] - m_fin)
        denom = smem[1] * sc + p_rc + p_s
        ctx = (acc_s[...] * sc + p_rc * rc_ref[...] + p_s * hs_s[...]) / denom
        att = _dot(_dot(ctx, wv[...]) + bv[...], wo[...]) + bo[...] + hs_s[...]
        out_ref[...] = jnp.maximum(att, 0.0)


def _att(hmix, s_sum, psk, patt, rc_init, s_init):
    wspec = pl.BlockSpec((H, H), lambda i: (0, 0))
    bspec = pl.BlockSpec((1, H), lambda i: (0, 0))
    return pl.pallas_call(
        _att_body,
        grid=(GRID,),
        in_specs=[
            pl.BlockSpec((RB, H), lambda i: (i, 0)),
            bspec, bspec, bspec,
            wspec,
            pl.BlockSpec((H, 3 * H), lambda i: (0, 0)),
            pl.BlockSpec((H, 3 * H), lambda i: (0, 0)),
            pl.BlockSpec((1, 3 * H), lambda i: (0, 0)),
            pl.BlockSpec((1, 3 * H), lambda i: (0, 0)),
            wspec, bspec, wspec, bspec, wspec, bspec, wspec, bspec,
        ],
        out_specs=pl.BlockSpec((1, H), lambda i: (0, 0)),
        out_shape=jax.ShapeDtypeStruct((1, H), jnp.float32),
        scratch_shapes=[
            pltpu.VMEM((1, H), jnp.float32),
            pltpu.VMEM((1, H), jnp.float32),
            pltpu.VMEM((1, H), jnp.float32),
            pltpu.SMEM((8,), jnp.float32),
        ],
    )(hmix, s_sum, rc_init, s_init, psk["W"], psk["Wih"], psk["Whh"],
      psk["bih"].reshape(1, -1), psk["bhh"].reshape(1, -1),
      patt["Wq"], patt["bq"].reshape(1, -1), patt["Wk"],
      patt["bk"].reshape(1, -1), patt["Wv"], patt["bv"].reshape(1, -1),
      patt["Wo"], patt["bo"].reshape(1, -1))


def kernel(x, edge_index, edge_attr, rc_mask, params):
    src = edge_index[0]
    dst = edge_index[1]
    pad = EPAD - E
    src3 = jnp.concatenate([src, jnp.zeros((pad,), jnp.int32)]).reshape(NT, NBLK, EB)
    trash = N + (jnp.arange(pad, dtype=jnp.int32) % (NACC - N))
    dst3 = jnp.concatenate([dst, trash]).reshape(NT, NBLK, EB)
    ea3 = jnp.concatenate([edge_attr, jnp.zeros((pad, EH), jnp.float32)]
                          ).reshape(NT, NBLK * EB, EH)
    zeros_h = jnp.zeros((NACC, H), jnp.float32)
    zeros_e = jnp.zeros((NACC, EH), jnp.float32)
    mask_f = (~rc_mask).astype(jnp.float32).reshape(N, 1)

    _apply_A, _edge_agg = _sc_kernels()
    eagg = _edge_agg(ea3, dst3, zeros_e)
    ax = _apply_A(x, src3, dst3, zeros_h)
    h1 = _dmpnn(ax, eagg, x, params["gnn1"])
    ah1 = _apply_A(h1, src3, dst3, zeros_h)
    h2 = _dmpnn(ah1, eagg, h1, params["gnn2"])
    ah2 = _apply_A(h2, src3, dst3, zeros_h)
    convs = params["mix"]["convs"]
    t1, s2, s3 = _gin3(h2, ah2, convs)
    as2 = _apply_A(s2, src3, dst3, zeros_h)
    as3 = _apply_A(s3, src3, dst3, zeros_h)
    t2, s3p = _gin2(s2, as2, s3, as3, convs[1], convs[2])
    as3p = _apply_A(s3p, src3, dst3, zeros_h)
    hmix, s_sum = _mix(t1, t2, s3p, as3p, convs[2], params["mix"],
                       params["skip"], mask_f)
    out = _att(hmix, s_sum, params["skip"], params["att"],
               params["rc_init"], params["s_init"])
    return out.reshape(H)
